# baseline (speedup 1.0000x reference)
import jax
import jax.numpy as jnp
from jax import lax
from jax.experimental import pallas as pl
from jax.experimental.pallas import tpu as pltpu

_S, _N, _DIN, _H, _KP, _L, _NH = 4, 2048, 128, 128, 128, 2, 4
_DH = _H // _NH
_FF = 4 * _H
_F32 = jnp.float32


def _layer_norm(x, g, b):
    m = jnp.mean(x, axis=-1, keepdims=True)
    v = jnp.mean((x - m) * (x - m), axis=-1, keepdims=True)
    return (x - m) * lax.rsqrt(v + 1e-5) * g + b


def _dot(a, b):
    return jnp.dot(a, b, preferred_element_type=_F32)


def _encoder_body(x_ref, idc_ref, idr_ref, win_ref, bin_ref,
                  ln1g_ref, ln1b_ref, wqk_ref, bqk_ref,
                  wvrep_ref, bvrep_ref, wo_ref, bo_ref, ln2g_ref, ln2b_ref,
                  w1_ref, c1_ref, w2_ref, c2_ref,
                  m1t_ref, dsum_ref,
                  hn_ref, sm_ref):
    x = x_ref[0]
    h = _dot(x, win_ref[...]) + bin_ref[...]

    idc = idc_ref[0]
    idr = idr_ref[0]
    lane_iota = lax.broadcasted_iota(jnp.int32, (_N, _KP), 1)
    A = (idc == lane_iota).astype(_F32)
    sub_iota = lax.broadcasted_iota(jnp.int32, (_KP, _N), 0)
    AT = (idr == sub_iota).astype(_F32)

    M1T = m1t_ref[...]
    Dsum = dsum_ref[...]

    for l in range(_L):
        y = _layer_norm(h, ln1g_ref[l], ln1b_ref[l])
        qk = _dot(y, wqk_ref[l]) + bqk_ref[l]
        pqk = jnp.where(qk > 0, qk + 1.0, jnp.exp(qk))
        pq, pk = pqk[:, :_H], pqk[:, _H:]

        ksum = _dot(AT, pk)

        v_rep_all = _dot(y, wvrep_ref[l]) + bvrep_ref[l]
        kvs = []
        for nh in range(_NH):
            sl = slice(nh * _DH, (nh + 1) * _DH)
            sl_rep = slice(nh * _DH * _DH, (nh + 1) * _DH * _DH)
            pk_n = pk[:, sl]
            pk_t = jnp.concatenate([pk_n] * _DH, axis=1)
            outer = pk_t * v_rep_all[:, sl_rep]
            kvs.append(_dot(AT, outer))
        kv_cat = jnp.concatenate(kvs + [ksum], axis=1)
        giant = _dot(A, kv_cat)
        kst = giant[:, _NH * _DH * _DH:]
        den_h = _dot(pq * kst, Dsum) + 1e-6

        heads = []
        for nh in range(_NH):
            sl = slice(nh * _DH, (nh + 1) * _DH)
            sl_rep = slice(nh * _DH * _DH, (nh + 1) * _DH * _DH)
            pq_n = pq[:, sl]
            kvt = giant[:, sl_rep]
            pq_t = jnp.concatenate([pq_n] * _DH, axis=1)
            z = pq_t * kvt
            num = _dot(z, M1T)
            heads.append(num / den_h[:, nh:nh + 1])
        attn = jnp.concatenate(heads, axis=1)
        h = h + _dot(attn, wo_ref[l]) + bo_ref[l]

        y2 = _layer_norm(h, ln2g_ref[l], ln2b_ref[l])
        ff = jnp.maximum(_dot(y2, w1_ref[l]) + c1_ref[l], 0.0)
        h = h + _dot(ff, w2_ref[l]) + c2_ref[l]

    hn_ref[0] = h
    counts = jnp.sum(AT, axis=1, keepdims=True)
    sums = _dot(AT, h)
    sm_ref[0] = sums / jnp.maximum(counts, 1.0)


def kernel(X, patch_ids, W_in, b_in, ln1_g, ln1_b, Wq, bq, Wk, bk, Wv, bv,
           Wo, bo, ln2_g, ln2_b, W1, c1, W2, c2):
    ids_i = patch_ids.astype(jnp.int32)
    ids_col = ids_i.reshape(_S, _N, 1)
    ids_row = ids_i.reshape(_S, 1, _N)

    eye = jnp.eye(_DH, dtype=_F32)
    M1 = jnp.kron(eye, jnp.ones((1, _DH), _F32))
    M1T = M1.T
    Rexp = jnp.kron(jnp.eye(_NH, dtype=_F32), M1)
    Wv_rep = jnp.einsum('lhk,kj->lhj', Wv, Rexp)
    bv_rep = jnp.einsum('lk,kj->lj', bv, Rexp).reshape(_L, 1, _NH * _DH * _DH)
    Dsum = jnp.kron(jnp.eye(_NH, dtype=_F32), jnp.ones((_DH, 1), _F32))

    Wqk = jnp.concatenate([Wq, Wk], axis=2)
    bqk = jnp.concatenate([bq, bk], axis=1).reshape(_L, 1, 2 * _H)
    b_in2 = b_in.reshape(1, _H)
    ln1_g3 = ln1_g.reshape(_L, 1, _H)
    ln1_b3 = ln1_b.reshape(_L, 1, _H)
    bo3 = bo.reshape(_L, 1, _H)
    ln2_g3 = ln2_g.reshape(_L, 1, _H)
    ln2_b3 = ln2_b.reshape(_L, 1, _H)
    c13 = c1.reshape(_L, 1, _FF)
    c23 = c2.reshape(_L, 1, _H)

    def rep2(shape):
        return pl.BlockSpec(shape, lambda s: (0,) * len(shape))

    def per_s(shape):
        return pl.BlockSpec(shape, lambda s: (s,) + (0,) * (len(shape) - 1))

    in_specs = [
        per_s((1, _N, _DIN)),
        per_s((1, _N, 1)),
        per_s((1, 1, _N)),
        rep2((_DIN, _H)),
        rep2((1, _H)),
        rep2((_L, 1, _H)),
        rep2((_L, 1, _H)),
        rep2((_L, _H, 2 * _H)),
        rep2((_L, 1, 2 * _H)),
        rep2((_L, _H, _NH * _DH * _DH)),
        rep2((_L, 1, _NH * _DH * _DH)),
        rep2((_L, _H, _H)),
        rep2((_L, 1, _H)),
        rep2((_L, 1, _H)),
        rep2((_L, 1, _H)),
        rep2((_L, _H, _FF)),
        rep2((_L, 1, _FF)),
        rep2((_L, _FF, _H)),
        rep2((_L, 1, _H)),
        rep2((_DH * _DH, _DH)),
        rep2((_H, _NH)),
    ]
    out_specs = [per_s((1, _N, _H)), per_s((1, _KP, _H))]
    out_shape = [
        jax.ShapeDtypeStruct((_S, _N, _H), _F32),
        jax.ShapeDtypeStruct((_S, _KP, _H), _F32),
    ]

    h_nodes, summaries = pl.pallas_call(
        _encoder_body,
        grid=(_S,),
        in_specs=in_specs,
        out_specs=out_specs,
        out_shape=out_shape,
        compiler_params=pltpu.CompilerParams(
            dimension_semantics=("parallel",)),
    )(X, ids_col, ids_row, W_in, b_in2, ln1_g3, ln1_b3, Wqk, bqk,
      Wv_rep, bv_rep, Wo, bo3, ln2_g3, ln2_b3, W1, c13, W2, c23, M1T, Dsum)
    return h_nodes, summaries


# speedup vs baseline: 1.0298x; 1.0298x over previous
import jax
import jax.numpy as jnp
from jax import lax
from jax.experimental import pallas as pl
from jax.experimental.pallas import tpu as pltpu

_S, _N, _DIN, _H, _KP, _L, _NH = 4, 2048, 128, 128, 128, 2, 4
_DH = _H // _NH
_FF = 4 * _H
_F32 = jnp.float32


def _layer_norm(x, g, b):
    m = jnp.mean(x, axis=-1, keepdims=True)
    v = jnp.mean((x - m) * (x - m), axis=-1, keepdims=True)
    return (x - m) * lax.rsqrt(v + 1e-5) * g + b


def _dot(a, b):
    return jnp.dot(a, b, preferred_element_type=_F32)


def _encoder_body(x_ref, idc_ref, idr_ref, win_ref, bin_ref,
                  ln1g_ref, ln1b_ref, wq_ref, bq_ref, wk_ref, bk_ref,
                  wvrep_ref, bvrep_ref, wo_ref, bo_ref, ln2g_ref, ln2b_ref,
                  w1_ref, c1_ref, w2_ref, c2_ref,
                  m1t_ref, dsum_ref,
                  hn_ref, sm_ref):
    x = x_ref[0]
    h = _dot(x, win_ref[...]) + bin_ref[...]

    idc = idc_ref[0]
    idr = idr_ref[0]
    lane_iota = lax.broadcasted_iota(jnp.int32, (_N, _KP), 1)
    A = (idc == lane_iota).astype(_F32)
    sub_iota = lax.broadcasted_iota(jnp.int32, (_KP, _N), 0)
    AT = (idr == sub_iota).astype(_F32)

    M1T = m1t_ref[...]
    Dsum = dsum_ref[...]

    for l in range(_L):
        y = _layer_norm(h, ln1g_ref[l], ln1b_ref[l])
        q = _dot(y, wq_ref[l]) + bq_ref[l]
        k = _dot(y, wk_ref[l]) + bk_ref[l]
        pq = jnp.where(q > 0, q + 1.0, jnp.exp(q))
        pk = jnp.where(k > 0, k + 1.0, jnp.exp(k))

        ksum = _dot(AT, pk)

        v_rep_all = _dot(y, wvrep_ref[l]) + bvrep_ref[l]
        kvs = []
        for nh in range(_NH):
            sl = slice(nh * _DH, (nh + 1) * _DH)
            sl_rep = slice(nh * _DH * _DH, (nh + 1) * _DH * _DH)
            pk_n = pk[:, sl]
            pk_t = jnp.concatenate([pk_n] * _DH, axis=1)
            outer = pk_t * v_rep_all[:, sl_rep]
            kvs.append(_dot(AT, outer))
        kv_cat = jnp.concatenate(kvs + [ksum], axis=1)
        giant = _dot(A, kv_cat)
        kst = giant[:, _NH * _DH * _DH:]
        den_h = _dot(pq * kst, Dsum) + 1e-6

        heads = []
        for nh in range(_NH):
            sl = slice(nh * _DH, (nh + 1) * _DH)
            sl_rep = slice(nh * _DH * _DH, (nh + 1) * _DH * _DH)
            pq_n = pq[:, sl]
            kvt = giant[:, sl_rep]
            pq_t = jnp.concatenate([pq_n] * _DH, axis=1)
            z = pq_t * kvt
            num = _dot(z, M1T)
            heads.append(num / den_h[:, nh:nh + 1])
        attn = jnp.concatenate(heads, axis=1)
        h = h + _dot(attn, wo_ref[l]) + bo_ref[l]

        y2 = _layer_norm(h, ln2g_ref[l], ln2b_ref[l])
        ff = jnp.maximum(_dot(y2, w1_ref[l]) + c1_ref[l], 0.0)
        h = h + _dot(ff, w2_ref[l]) + c2_ref[l]

    hn_ref[0] = h
    counts = jnp.sum(AT, axis=1, keepdims=True)
    sums = _dot(AT, h)
    sm_ref[0] = sums / jnp.maximum(counts, 1.0)


def kernel(X, patch_ids, W_in, b_in, ln1_g, ln1_b, Wq, bq, Wk, bk, Wv, bv,
           Wo, bo, ln2_g, ln2_b, W1, c1, W2, c2):
    ids_i = patch_ids.astype(jnp.int32)
    ids_col = ids_i.reshape(_S, _N, 1)
    ids_row = ids_i.reshape(_S, 1, _N)

    eye = jnp.eye(_DH, dtype=_F32)
    M1 = jnp.kron(eye, jnp.ones((1, _DH), _F32))
    M1T = M1.T
    Rexp = jnp.kron(jnp.eye(_NH, dtype=_F32), M1)
    Wv_rep = jnp.einsum('lhk,kj->lhj', Wv, Rexp)
    bv_rep = jnp.einsum('lk,kj->lj', bv, Rexp).reshape(_L, 1, _NH * _DH * _DH)
    Dsum = jnp.kron(jnp.eye(_NH, dtype=_F32), jnp.ones((_DH, 1), _F32))

    b_in2 = b_in.reshape(1, _H)
    ln1_g3 = ln1_g.reshape(_L, 1, _H)
    ln1_b3 = ln1_b.reshape(_L, 1, _H)
    bq3 = bq.reshape(_L, 1, _H)
    bk3 = bk.reshape(_L, 1, _H)
    bo3 = bo.reshape(_L, 1, _H)
    ln2_g3 = ln2_g.reshape(_L, 1, _H)
    ln2_b3 = ln2_b.reshape(_L, 1, _H)
    c13 = c1.reshape(_L, 1, _FF)
    c23 = c2.reshape(_L, 1, _H)

    def rep2(shape):
        return pl.BlockSpec(shape, lambda s: (0,) * len(shape))

    def per_s(shape):
        return pl.BlockSpec(shape, lambda s: (s,) + (0,) * (len(shape) - 1))

    in_specs = [
        per_s((1, _N, _DIN)),
        per_s((1, _N, 1)),
        per_s((1, 1, _N)),
        rep2((_DIN, _H)),
        rep2((1, _H)),
        rep2((_L, 1, _H)),
        rep2((_L, 1, _H)),
        rep2((_L, _H, _H)),
        rep2((_L, 1, _H)),
        rep2((_L, _H, _H)),
        rep2((_L, 1, _H)),
        rep2((_L, _H, _NH * _DH * _DH)),
        rep2((_L, 1, _NH * _DH * _DH)),
        rep2((_L, _H, _H)),
        rep2((_L, 1, _H)),
        rep2((_L, 1, _H)),
        rep2((_L, 1, _H)),
        rep2((_L, _H, _FF)),
        rep2((_L, 1, _FF)),
        rep2((_L, _FF, _H)),
        rep2((_L, 1, _H)),
        rep2((_DH * _DH, _DH)),
        rep2((_H, _NH)),
    ]
    out_specs = [per_s((1, _N, _H)), per_s((1, _KP, _H))]
    out_shape = [
        jax.ShapeDtypeStruct((_S, _N, _H), _F32),
        jax.ShapeDtypeStruct((_S, _KP, _H), _F32),
    ]

    h_nodes, summaries = pl.pallas_call(
        _encoder_body,
        grid=(_S,),
        in_specs=in_specs,
        out_specs=out_specs,
        out_shape=out_shape,
        compiler_params=pltpu.CompilerParams(
            dimension_semantics=("parallel",)),
    )(X, ids_col, ids_row, W_in, b_in2, ln1_g3, ln1_b3, Wq, bq3, Wk, bk3,
      Wv_rep, bv_rep, Wo, bo3, ln2_g3, ln2_b3, W1, c13, W2, c23, M1T, Dsum)
    return h_nodes, summaries
